# baseline (speedup 1.0000x reference)
import functools

import jax
import jax.numpy as jnp
from jax import lax
from jax.experimental import pallas as pl
from jax.experimental.pallas import tpu as pltpu
from jax.experimental.pallas import tpu_sc as plsc

_NC = 2
_NS = 16
_NW = _NC * _NS
_C = 256
_D = 128


def _splat(v):
    return jnp.full((16,), v, jnp.int32)


def _sc_body(x_hbm, sw_hbm, mw_hbm, hw_hbm, dw_hbm, mow_hbm, out_hbm,
             xv, t2a, t2b, emo, esw, emw, ehw, edw, ob):
    wid = lax.axis_index("s") * _NC + lax.axis_index("c")
    t_total = out_hbm.shape[0]
    per_w = t_total // _NW
    n_chunks = per_w // _C
    base_w = wid * per_w

    pltpu.sync_copy(sw_hbm, esw)
    pltpu.sync_copy(mw_hbm, emw)
    pltpu.sync_copy(hw_hbm, ehw)
    pltpu.sync_copy(dw_hbm, edw)
    pltpu.sync_copy(mow_hbm, emo)

    def build_a(a, _):
        def build_b(b, _):
            r = a * 13 + b
            for j in range(_D // 16):
                s = pl.ds(j * 16, 16)
                t2a[r, s] = esw[a, s] + emw[b, s]
                t2b[r, s] = ehw[a, s] + edw[b, s]
            return 0
        return lax.fori_loop(0, 13, build_b, 0)

    lax.fori_loop(0, 13, build_a, 0)

    def do_chunk(ci, _):
        base = base_w + ci * _C
        pltpu.sync_copy(x_hbm.at[pl.ds(base * 5, _C * 5)], xv.at[pl.ds(0, _C * 5)])

        def do_token(t, _):
            v = xv[pl.ds(t * 5, 16)]
            mo = v[0]
            dd = v[1]
            hh = v[2]
            mi = v[3]
            ss = v[4]
            k1 = ss * 13 + mi
            k2 = hh * 13 + dd
            for j in range(_D // 16):
                s = pl.ds(j * 16, 16)
                ob[t, s] = t2a[k1, s] + t2b[k2, s] + emo[mo, s]
            return 0

        lax.fori_loop(0, _C, do_token, 0)
        pltpu.sync_copy(ob, out_hbm.at[pl.ds(base, _C)])
        return 0

    lax.fori_loop(0, n_chunks, do_chunk, 0)


def kernel(x, second_w, minute_w, hour_w, day_w, month_w):
    b, s, _ = x.shape
    t = b * s
    xf = x.reshape(t * 5)

    mesh = plsc.VectorSubcoreMesh(core_axis_name="c", subcore_axis_name="s")
    run = functools.partial(
        pl.kernel,
        mesh=mesh,
        out_type=jax.ShapeDtypeStruct((t, _D), jnp.float32),
        scratch_types=[
            pltpu.VMEM((_C * 5 + 16,), jnp.int32),
            pltpu.VMEM((176, _D), jnp.float32),
            pltpu.VMEM((176, _D), jnp.float32),
            pltpu.VMEM((16, _D), jnp.float32),
            pltpu.VMEM((16, _D), jnp.float32),
            pltpu.VMEM((16, _D), jnp.float32),
            pltpu.VMEM((16, _D), jnp.float32),
            pltpu.VMEM((16, _D), jnp.float32),
            pltpu.VMEM((_C, _D), jnp.float32),
        ],
    )(_sc_body)
    def pad16(w):
        return jnp.zeros((16, _D), w.dtype).at[:13].set(w[:13])

    out = run(xf, pad16(second_w), pad16(minute_w), pad16(hour_w),
              pad16(day_w), pad16(month_w))
    return out.reshape(b, s, _D)


# speedup vs baseline: 2.6404x; 2.6404x over previous
import functools

import jax
import jax.numpy as jnp
from jax import lax
from jax.experimental import pallas as pl
from jax.experimental.pallas import tpu as pltpu
from jax.experimental.pallas import tpu_sc as plsc

_NC = 2
_NS = 16
_NW = _NC * _NS
_C = 256
_D = 128
_T_SC = 73728
_BLK = 53248


def _sc_body(x_hbm, sw_hbm, mw_hbm, hw_hbm, dw_hbm, mow_hbm, out_hbm,
             xv, t2a, t2b, emo, esw, emw, ehw, edw, ob):
    wid = lax.axis_index("s") * _NC + lax.axis_index("c")
    t_total = out_hbm.shape[0]
    per_w = t_total // _NW
    n_chunks = per_w // _C
    base_w = wid * per_w

    pltpu.sync_copy(sw_hbm, esw)
    pltpu.sync_copy(mw_hbm, emw)
    pltpu.sync_copy(hw_hbm, ehw)
    pltpu.sync_copy(dw_hbm, edw)
    pltpu.sync_copy(mow_hbm, emo)

    def build_a(a, _):
        def build_b(b, _):
            r = a * 13 + b
            for j in range(_D // 16):
                s = pl.ds(j * 16, 16)
                t2a[r, s] = esw[a, s] + emw[b, s]
                t2b[r, s] = ehw[a, s] + edw[b, s]
            return 0
        return lax.fori_loop(0, 13, build_b, 0)

    lax.fori_loop(0, 13, build_a, 0)

    def do_chunk(ci, _):
        base = base_w + ci * _C
        pltpu.sync_copy(x_hbm.at[pl.ds(base * 5, _C * 5)], xv.at[pl.ds(0, _C * 5)])

        def do_token(t, _):
            v = xv[pl.ds(t * 5, 16)]
            mo = v[0]
            dd = v[1]
            hh = v[2]
            mi = v[3]
            ss = v[4]
            k1 = ss * 13 + mi
            k2 = hh * 13 + dd
            for j in range(_D // 16):
                s = pl.ds(j * 16, 16)
                ob[t, s] = t2a[k1, s] + t2b[k2, s] + emo[mo, s]
            return 0

        lax.fori_loop(0, _C, do_token, 0)
        pltpu.sync_copy(ob, out_hbm.at[pl.ds(base, _C)])
        return 0

    lax.fori_loop(0, n_chunks, do_chunk, 0)


def _tc_body(xt_ref, w_ref, out_ref):
    n = xt_ref.shape[1]
    iota16 = lax.broadcasted_iota(jnp.int32, (16, n), 0)
    cols = []
    for f in range(5):
        row = xt_ref[f : f + 1, :]
        cols.append((row == iota16).astype(jnp.float32))
    m = jnp.concatenate(cols, axis=0)
    out_ref[...] = lax.dot_general(
        m, w_ref[...], (((0,), (0,)), ((), ())), preferred_element_type=jnp.float32
    )


def _run_sc(xflat, tabs16):
    mesh = plsc.VectorSubcoreMesh(core_axis_name="c", subcore_axis_name="s")
    run = functools.partial(
        pl.kernel,
        mesh=mesh,
        out_type=jax.ShapeDtypeStruct((_T_SC, _D), jnp.float32),
        scratch_types=[
            pltpu.VMEM((_C * 5 + 16,), jnp.int32),
            pltpu.VMEM((176, _D), jnp.float32),
            pltpu.VMEM((176, _D), jnp.float32),
            pltpu.VMEM((16, _D), jnp.float32),
            pltpu.VMEM((16, _D), jnp.float32),
            pltpu.VMEM((16, _D), jnp.float32),
            pltpu.VMEM((16, _D), jnp.float32),
            pltpu.VMEM((16, _D), jnp.float32),
            pltpu.VMEM((_C, _D), jnp.float32),
        ],
    )(_sc_body)
    sw, mw, hw, dw, mow = tabs16
    return run(xflat, sw, mw, hw, dw, mow)


def _run_tc(xt, w80, n_tc):
    grid = n_tc // _BLK
    return pl.pallas_call(
        _tc_body,
        grid=(grid,),
        in_specs=[
            pl.BlockSpec((5, _BLK), lambda i: (0, i)),
            pl.BlockSpec((80, _D), lambda i: (0, 0)),
        ],
        out_specs=pl.BlockSpec((_BLK, _D), lambda i: (i, 0)),
        out_shape=jax.ShapeDtypeStruct((n_tc, _D), jnp.float32),
    )(xt, w80)


def kernel(x, second_w, minute_w, hour_w, day_w, month_w):
    b, s, _ = x.shape
    t = b * s
    n_tc = t - _T_SC
    xf = x.reshape(t, 5)

    def pad16(w):
        return jnp.zeros((16, _D), w.dtype).at[:13].set(w[:13])

    tabs = (second_w, minute_w, hour_w, day_w, month_w)
    w80 = jnp.zeros((80, _D), jnp.float32)
    for f, tab in enumerate((month_w, day_w, hour_w, minute_w, second_w)):
        w80 = w80.at[16 * f : 16 * f + 13].set(tab[:13])

    out_sc = _run_sc(x.reshape(t * 5), tuple(pad16(w) for w in tabs))
    xt_tc = xf[_T_SC:].T
    out_tc = _run_tc(xt_tc, w80, n_tc)
    out = jnp.concatenate([out_sc, out_tc], axis=0)
    return out.reshape(b, s, _D)


# speedup vs baseline: 2.9075x; 1.1012x over previous
import functools

import jax
import jax.numpy as jnp
from jax import lax
from jax.experimental import pallas as pl
from jax.experimental.pallas import tpu as pltpu
from jax.experimental.pallas import tpu_sc as plsc

_NC = 2
_NS = 16
_NW = _NC * _NS
_C = 256
_D = 128


def _sc_body(x_hbm, sw_hbm, mw_hbm, hw_hbm, dw_hbm, mow_hbm, out_hbm,
             xva, xvb, t2a, t2b, emo, esw, emw, ehw, edw, oba, obb,
             sxa, sxb, soa, sob):
    wid = lax.axis_index("s") * _NC + lax.axis_index("c")
    t_total = out_hbm.shape[0]
    per_w = t_total // _NW
    n_chunks = per_w // _C
    base_w = wid * per_w

    pltpu.sync_copy(sw_hbm, esw)
    pltpu.sync_copy(mw_hbm, emw)
    pltpu.sync_copy(hw_hbm, ehw)
    pltpu.sync_copy(dw_hbm, edw)
    pltpu.sync_copy(mow_hbm, emo)

    def build_a(a, _):
        def build_b(b, _):
            r = a * 13 + b
            for j in range(_D // 16):
                s = pl.ds(j * 16, 16)
                t2a[r, s] = esw[a, s] + emw[b, s]
                t2b[r, s] = ehw[a, s] + edw[b, s]
            return 0
        return lax.fori_loop(0, 13, build_b, 0)

    lax.fori_loop(0, 13, build_a, 0)

    def x_in(ci, xv, sem):
        return pltpu.make_async_copy(
            x_hbm.at[pl.ds((base_w + ci * _C) * 5, _C * 5)],
            xv.at[pl.ds(0, _C * 5)], sem)

    def o_out(ci, ob, sem):
        return pltpu.make_async_copy(
            ob, out_hbm.at[pl.ds(base_w + ci * _C, _C)], sem)

    def compute_chunk(xv, ob):
        @plsc.parallel_loop(0, _C, unroll=4)
        def tok(t):
            v = xv[pl.ds(t * 5, 16)]
            mo = v[0]
            dd = v[1]
            hh = v[2]
            mi = v[3]
            ss = v[4]
            k1 = ss * 13 + mi
            k2 = hh * 13 + dd
            for j in range(_D // 16):
                s = pl.ds(j * 16, 16)
                ob[t, s] = t2a[k1, s] + t2b[k2, s] + emo[mo, s]

    x_in(0, xva, sxa).start()

    def iter2(i, _):
        c0 = i * 2
        x_in(c0 + 1, xvb, sxb).start()
        x_in(c0, xva, sxa).wait()

        @pl.when(i > 0)
        def _():
            o_out(c0 - 2, oba, soa).wait()

        compute_chunk(xva, oba)
        o_out(c0, oba, soa).start()

        @pl.when(c0 + 2 < n_chunks)
        def _():
            x_in(c0 + 2, xva, sxa).start()

        x_in(c0 + 1, xvb, sxb).wait()

        @pl.when(i > 0)
        def _():
            o_out(c0 - 1, obb, sob).wait()

        compute_chunk(xvb, obb)
        o_out(c0 + 1, obb, sob).start()
        return 0

    lax.fori_loop(0, n_chunks // 2, iter2, 0)
    o_out(n_chunks - 2, oba, soa).wait()
    o_out(n_chunks - 1, obb, sob).wait()


def kernel(x, second_w, minute_w, hour_w, day_w, month_w):
    b, s, _ = x.shape
    t = b * s
    xf = x.reshape(t * 5)

    mesh = plsc.VectorSubcoreMesh(core_axis_name="c", subcore_axis_name="s")
    run = functools.partial(
        pl.kernel,
        mesh=mesh,
        out_type=jax.ShapeDtypeStruct((t, _D), jnp.float32),
        scratch_types=[
            pltpu.VMEM((_C * 5 + 16,), jnp.int32),
            pltpu.VMEM((_C * 5 + 16,), jnp.int32),
            pltpu.VMEM((176, _D), jnp.float32),
            pltpu.VMEM((176, _D), jnp.float32),
            pltpu.VMEM((16, _D), jnp.float32),
            pltpu.VMEM((16, _D), jnp.float32),
            pltpu.VMEM((16, _D), jnp.float32),
            pltpu.VMEM((16, _D), jnp.float32),
            pltpu.VMEM((16, _D), jnp.float32),
            pltpu.VMEM((_C, _D), jnp.float32),
            pltpu.VMEM((_C, _D), jnp.float32),
            pltpu.SemaphoreType.DMA,
            pltpu.SemaphoreType.DMA,
            pltpu.SemaphoreType.DMA,
            pltpu.SemaphoreType.DMA,
        ],
    )(_sc_body)

    def pad16(w):
        return jnp.zeros((16, _D), w.dtype).at[:13].set(w[:13])

    out = run(xf, pad16(second_w), pad16(minute_w), pad16(hour_w),
              pad16(day_w), pad16(month_w))
    return out.reshape(b, s, _D)
